# baseline (device time: 26216 ns/iter reference)
import jax
import jax.numpy as jnp
from jax import lax
from jax.experimental import pallas as pl
from jax.experimental.pallas import tpu as pltpu

N_DEV = 4
M_PER = 512
D = 512
DH = D // 2


def kernel(partial, gamma):
    x = partial.reshape(N_DEV * M_PER, D)
    g = gamma.reshape(1, D)

    def body(x_ref, g_ref, out_ref,
             recv_a1, recv_b1, send_a2, send_b2, recv_a2, recv_b2,
             vx_own, vx_fa, vx_fb, vg,
             sems_send_a, sems_recv_a, sems_send_b, sems_recv_b,
             sems_local):
        my = lax.axis_index("i")
        left = lax.rem(my + N_DEV - 1, N_DEV)
        right = lax.rem(my + 1, N_DEV)
        q = my ^ 1
        r = 3 - my

        def row(c):
            return pl.ds(c * M_PER, M_PER)

        cp_own = pltpu.make_async_copy(
            x_ref.at[row(my), :], vx_own, sems_local.at[0])
        cp_fa = pltpu.make_async_copy(
            x_ref.at[row(3 - my), pl.ds(0, DH)], vx_fa, sems_local.at[1])
        cp_fb = pltpu.make_async_copy(
            x_ref.at[row(q), pl.ds(DH, DH)], vx_fb, sems_local.at[2])
        cp_g = pltpu.make_async_copy(g_ref, vg, sems_local.at[3])
        cp_fa.start()
        cp_fb.start()
        cp_own.start()
        cp_g.start()

        barrier_sem = pltpu.get_barrier_semaphore()
        for nbr in (left, right):
            pl.semaphore_signal(
                barrier_sem, inc=1,
                device_id=(nbr,), device_id_type=pl.DeviceIdType.MESH,
            )
        pl.semaphore_wait(barrier_sem, 2)

        a1 = []
        for k, c in enumerate((3 - q, q)):
            a1.append(pltpu.make_async_remote_copy(
                src_ref=x_ref.at[row(c), pl.ds(0, DH)],
                dst_ref=recv_a1.at[k],
                send_sem=sems_send_a.at[k],
                recv_sem=sems_recv_a.at[k],
                device_id=(q,),
                device_id_type=pl.DeviceIdType.MESH,
            ))
        b1 = []
        for k, c in enumerate((r ^ 1, r)):
            b1.append(pltpu.make_async_remote_copy(
                src_ref=x_ref.at[row(c), pl.ds(DH, DH)],
                dst_ref=recv_b1.at[k],
                send_sem=sems_send_b.at[k],
                recv_sem=sems_recv_b.at[k],
                device_id=(r,),
                device_id_type=pl.DeviceIdType.MESH,
            ))
        a1[0].start()
        b1[0].start()
        a1[1].start()
        b1[1].start()

        a1[0].wait_recv()
        cp_fa.wait()
        send_a2[:, :] = recv_a1[0] + vx_fa[:, :]
        a2 = pltpu.make_async_remote_copy(
            src_ref=send_a2,
            dst_ref=recv_a2,
            send_sem=sems_send_a.at[2],
            recv_sem=sems_recv_a.at[2],
            device_id=(r,),
            device_id_type=pl.DeviceIdType.MESH,
        )
        a2.start()
        b1[0].wait_recv()
        cp_fb.wait()
        send_b2[:, :] = recv_b1[0] + vx_fb[:, :]
        b2 = pltpu.make_async_remote_copy(
            src_ref=send_b2,
            dst_ref=recv_b2,
            send_sem=sems_send_b.at[2],
            recv_sem=sems_recv_b.at[2],
            device_id=(q,),
            device_id_type=pl.DeviceIdType.MESH,
        )
        b2.start()

        a1[1].wait_recv()
        cp_own.wait()
        recv_a1[0, :, :] = recv_a1[1] + vx_own[:, pl.ds(0, DH)]
        b1[1].wait_recv()
        recv_b1[0, :, :] = recv_b1[1] + vx_own[:, pl.ds(DH, DH)]
        cp_g.wait()
        a2.wait_recv()
        y_a = recv_a1[0] + recv_a2[:, :]
        b2.wait_recv()
        y_b = recv_b1[0] + recv_b2[:, :]
        ssq = (jnp.sum(y_a * y_a, axis=-1, keepdims=True)
               + jnp.sum(y_b * y_b, axis=-1, keepdims=True))
        scale = lax.rsqrt(ssq / D + 1e-6)
        out_ref[:, pl.ds(0, DH)] = y_a * scale * vg[:, pl.ds(0, DH)]
        out_ref[:, pl.ds(DH, DH)] = y_b * scale * vg[:, pl.ds(DH, DH)]

        for d in (a1[0], a1[1], b1[0], b1[1], a2, b2):
            d.wait_send()

    return pl.pallas_call(
        body,
        out_shape=jax.ShapeDtypeStruct((M_PER, D), jnp.float32),
        in_specs=[
            pl.BlockSpec(memory_space=pl.ANY),
            pl.BlockSpec(memory_space=pl.ANY),
        ],
        out_specs=pl.BlockSpec(memory_space=pltpu.VMEM),
        scratch_shapes=[
            pltpu.VMEM((2, M_PER, DH), jnp.float32),
            pltpu.VMEM((2, M_PER, DH), jnp.float32),
            pltpu.VMEM((M_PER, DH), jnp.float32),
            pltpu.VMEM((M_PER, DH), jnp.float32),
            pltpu.VMEM((M_PER, DH), jnp.float32),
            pltpu.VMEM((M_PER, DH), jnp.float32),
            pltpu.VMEM((M_PER, D), jnp.float32),
            pltpu.VMEM((M_PER, DH), jnp.float32),
            pltpu.VMEM((M_PER, DH), jnp.float32),
            pltpu.VMEM((1, D), jnp.float32),
            pltpu.SemaphoreType.DMA((3,)),
            pltpu.SemaphoreType.DMA((3,)),
            pltpu.SemaphoreType.DMA((3,)),
            pltpu.SemaphoreType.DMA((3,)),
            pltpu.SemaphoreType.DMA((4,)),
        ],
        compiler_params=pltpu.CompilerParams(collective_id=0),
    )(x, g)


# device time: 23329 ns/iter; 1.1238x vs baseline; 1.1238x over previous
import jax
import jax.numpy as jnp
from jax import lax
from jax.experimental import pallas as pl
from jax.experimental.pallas import tpu as pltpu

N_DEV = 4
M_PER = 512
D = 512
DH = D // 2


def kernel(partial, gamma):
    x = partial.reshape(N_DEV * M_PER, D)
    g = gamma.reshape(1, D)
    x = pltpu.with_memory_space_constraint(x, pltpu.MemorySpace.HBM)
    g = pltpu.with_memory_space_constraint(g, pltpu.MemorySpace.HBM)

    def body(x_ref, g_ref, out_ref,
             recv_a1, recv_b1, send_a2, send_b2, recv_a2, recv_b2,
             vx_own, vx_fa, vx_fb, vg,
             sems_send_a, sems_recv_a, sems_send_b, sems_recv_b,
             sems_local):
        my = lax.axis_index("i")
        left = lax.rem(my + N_DEV - 1, N_DEV)
        right = lax.rem(my + 1, N_DEV)
        q = my ^ 1
        r = 3 - my

        def row(c):
            return pl.ds(c * M_PER, M_PER)

        cp_own = pltpu.make_async_copy(
            x_ref.at[row(my), :], vx_own, sems_local.at[0])
        cp_fa = pltpu.make_async_copy(
            x_ref.at[row(3 - my), pl.ds(0, DH)], vx_fa, sems_local.at[1])
        cp_fb = pltpu.make_async_copy(
            x_ref.at[row(q), pl.ds(DH, DH)], vx_fb, sems_local.at[2])
        cp_g = pltpu.make_async_copy(g_ref, vg, sems_local.at[3])
        cp_fa.start()
        cp_fb.start()
        cp_own.start()
        cp_g.start()

        barrier_sem = pltpu.get_barrier_semaphore()
        for nbr in (left, right):
            pl.semaphore_signal(
                barrier_sem, inc=1,
                device_id=(nbr,), device_id_type=pl.DeviceIdType.MESH,
            )
        pl.semaphore_wait(barrier_sem, 2)

        a1 = []
        for k, c in enumerate((3 - q, q)):
            a1.append(pltpu.make_async_remote_copy(
                src_ref=x_ref.at[row(c), pl.ds(0, DH)],
                dst_ref=recv_a1.at[k],
                send_sem=sems_send_a.at[k],
                recv_sem=sems_recv_a.at[k],
                device_id=(q,),
                device_id_type=pl.DeviceIdType.MESH,
            ))
        b1 = []
        for k, c in enumerate((r ^ 1, r)):
            b1.append(pltpu.make_async_remote_copy(
                src_ref=x_ref.at[row(c), pl.ds(DH, DH)],
                dst_ref=recv_b1.at[k],
                send_sem=sems_send_b.at[k],
                recv_sem=sems_recv_b.at[k],
                device_id=(r,),
                device_id_type=pl.DeviceIdType.MESH,
            ))
        a1[0].start()
        b1[0].start()
        a1[1].start()
        b1[1].start()

        a1[0].wait_recv()
        cp_fa.wait()
        send_a2[:, :] = recv_a1[0] + vx_fa[:, :]
        a2 = pltpu.make_async_remote_copy(
            src_ref=send_a2,
            dst_ref=recv_a2,
            send_sem=sems_send_a.at[2],
            recv_sem=sems_recv_a.at[2],
            device_id=(r,),
            device_id_type=pl.DeviceIdType.MESH,
        )
        a2.start()
        b1[0].wait_recv()
        cp_fb.wait()
        send_b2[:, :] = recv_b1[0] + vx_fb[:, :]
        b2 = pltpu.make_async_remote_copy(
            src_ref=send_b2,
            dst_ref=recv_b2,
            send_sem=sems_send_b.at[2],
            recv_sem=sems_recv_b.at[2],
            device_id=(q,),
            device_id_type=pl.DeviceIdType.MESH,
        )
        b2.start()

        a1[1].wait_recv()
        cp_own.wait()
        recv_a1[0, :, :] = recv_a1[1] + vx_own[:, pl.ds(0, DH)]
        b1[1].wait_recv()
        recv_b1[0, :, :] = recv_b1[1] + vx_own[:, pl.ds(DH, DH)]
        cp_g.wait()
        a2.wait_recv()
        y_a = recv_a1[0] + recv_a2[:, :]
        b2.wait_recv()
        y_b = recv_b1[0] + recv_b2[:, :]
        ssq = (jnp.sum(y_a * y_a, axis=-1, keepdims=True)
               + jnp.sum(y_b * y_b, axis=-1, keepdims=True))
        scale = lax.rsqrt(ssq / D + 1e-6)
        out_ref[:, pl.ds(0, DH)] = y_a * scale * vg[:, pl.ds(0, DH)]
        out_ref[:, pl.ds(DH, DH)] = y_b * scale * vg[:, pl.ds(DH, DH)]

        for d in (a1[0], a1[1], b1[0], b1[1], a2, b2):
            d.wait_send()

    return pl.pallas_call(
        body,
        out_shape=jax.ShapeDtypeStruct((M_PER, D), jnp.float32),
        in_specs=[
            pl.BlockSpec(memory_space=pl.ANY),
            pl.BlockSpec(memory_space=pl.ANY),
        ],
        out_specs=pl.BlockSpec(memory_space=pltpu.VMEM),
        scratch_shapes=[
            pltpu.VMEM((2, M_PER, DH), jnp.float32),
            pltpu.VMEM((2, M_PER, DH), jnp.float32),
            pltpu.VMEM((M_PER, DH), jnp.float32),
            pltpu.VMEM((M_PER, DH), jnp.float32),
            pltpu.VMEM((M_PER, DH), jnp.float32),
            pltpu.VMEM((M_PER, DH), jnp.float32),
            pltpu.VMEM((M_PER, D), jnp.float32),
            pltpu.VMEM((M_PER, DH), jnp.float32),
            pltpu.VMEM((M_PER, DH), jnp.float32),
            pltpu.VMEM((1, D), jnp.float32),
            pltpu.SemaphoreType.DMA((3,)),
            pltpu.SemaphoreType.DMA((3,)),
            pltpu.SemaphoreType.DMA((3,)),
            pltpu.SemaphoreType.DMA((3,)),
            pltpu.SemaphoreType.DMA((4,)),
        ],
        compiler_params=pltpu.CompilerParams(collective_id=0),
    )(x, g)
